# Initial kernel scaffold; baseline (speedup 1.0000x reference)
import jax
import jax.numpy as jnp
from jax.experimental import pallas as pl


def kernel(variant_types_b, allele_frequencies_b, haplotypes_bs, priors_vc, snv_log_priors_rrra):
    raise NotImplementedError("write your pallas kernel here")


# speedup vs baseline: 11.9068x; 11.9068x over previous
import functools

import jax
import jax.numpy as jnp
from jax import lax
from jax.experimental import pallas as pl
from jax.experimental.pallas import tpu as pltpu
from jax.experimental.pallas import tpu_sc as plsc

B = 524288
S2 = 26
NC = 2
NS = 16
NW = NC * NS
BPW = B // NW
CHUNK = 1024
NCHUNK = BPW // CHUNK
GROUPS = CHUNK // 16

_LN2 = 0.69314718055994530942


def _vlog(x):
    bits = plsc.bitcast(x, jnp.int32)
    e = lax.shift_right_logical(bits, 23) - 127
    mbits = (bits & 0x007FFFFF) | 0x3F800000
    m = plsc.bitcast(mbits, jnp.float32)
    big = m > 1.41421356
    m = jnp.where(big, m * 0.5, m)
    ef = e.astype(jnp.float32) + jnp.where(big, 1.0, 0.0)
    t = m - 1.0
    z = t * t
    p = jnp.full((16,), 7.0376836292e-2, jnp.float32)
    p = p * t + (-1.1514610310e-1)
    p = p * t + 1.1676998740e-1
    p = p * t + (-1.2420140846e-1)
    p = p * t + 1.4249322787e-1
    p = p * t + (-1.6668057665e-1)
    p = p * t + 2.0000714765e-1
    p = p * t + (-2.4999993993e-1)
    p = p * t + 3.3333331174e-1
    y = t * (z * p) - 0.5 * z + t
    return y + ef * _LN2


def _sc_body(vt_hbm, af_hbm, hap_hbm, pri_hbm, snv_hbm, out_hbm,
             vt_v, af_v, hap_v, out_v, pri_v, snv_v):
    wid = lax.axis_index("s") * NC + lax.axis_index("c")
    base_w = wid * BPW
    pltpu.sync_copy(pri_hbm, pri_v)
    pltpu.sync_copy(snv_hbm, snv_v)
    lanes0 = lax.iota(jnp.int32, 16)

    def chunk_body(ci, carry):
        cb = base_w + ci * CHUNK
        pltpu.sync_copy(vt_hbm.at[pl.ds(cb, CHUNK)], vt_v)
        pltpu.sync_copy(af_hbm.at[pl.ds(cb, CHUNK)], af_v)
        pltpu.sync_copy(hap_hbm.at[pl.ds(cb * S2, CHUNK * S2)], hap_v)

        def group_body(gi, gcarry):
            gb = gi * 16
            lanes = lanes0 + gb
            vt = vt_v[pl.ds(gb, 16)]
            af = af_v[pl.ds(gb, 16)]
            hrow = lanes * S2
            i0 = plsc.load_gather(hap_v, [hrow + 5])
            i1 = plsc.load_gather(hap_v, [hrow + 6])
            i2 = plsc.load_gather(hap_v, [hrow + 7])
            i3 = plsc.load_gather(hap_v, [hrow + 19])
            snv = plsc.load_gather(snv_v, [((i0 * 5 + i1) * 5 + i2) * 5 + i3])
            vt5 = vt * 5
            p_som = plsc.load_gather(pri_v, [vt5])
            p_art = plsc.load_gather(pri_v, [vt5 + 1])
            p_nart = plsc.load_gather(pri_v, [vt5 + 4])
            is_snv = vt == 0
            c0 = jnp.where(is_snv, snv, p_som)
            u = 1.0 - af
            c3 = _vlog(1.0 - u * u)
            m = jnp.maximum(jnp.maximum(c0, p_art),
                            jnp.maximum(jnp.maximum(c3, p_nart), 0.0))
            s = (jnp.exp(c0 - m) + jnp.exp(p_art - m) + jnp.exp(-m)
                 + jnp.exp(c3 - m) + jnp.exp(p_nart - m))
            lse = m + _vlog(s)
            orow = lanes * 5
            plsc.store_scatter(out_v, [orow], c0 - lse)
            plsc.store_scatter(out_v, [orow + 1], p_art - lse)
            plsc.store_scatter(out_v, [orow + 2], -lse)
            plsc.store_scatter(out_v, [orow + 3], c3 - lse)
            plsc.store_scatter(out_v, [orow + 4], p_nart - lse)
            return gcarry

        lax.fori_loop(0, GROUPS, group_body, 0)
        pltpu.sync_copy(out_v, out_hbm.at[pl.ds(cb * 5, CHUNK * 5)])
        return carry

    lax.fori_loop(0, NCHUNK, chunk_body, 0)


_sc_kernel = functools.partial(
    pl.kernel,
    mesh=plsc.VectorSubcoreMesh(core_axis_name="c", subcore_axis_name="s"),
    out_type=jax.ShapeDtypeStruct((B * 5,), jnp.float32),
    scratch_types=[
        pltpu.VMEM((CHUNK,), jnp.int32),
        pltpu.VMEM((CHUNK,), jnp.float32),
        pltpu.VMEM((CHUNK * S2,), jnp.int32),
        pltpu.VMEM((CHUNK * 5,), jnp.float32),
        pltpu.VMEM((25,), jnp.float32),
        pltpu.VMEM((625,), jnp.float32),
    ],
    compiler_params=pltpu.CompilerParams(needs_layout_passes=False),
)(_sc_body)


def kernel(variant_types_b, allele_frequencies_b, haplotypes_bs,
           priors_vc, snv_log_priors_rrra):
    out_flat = _sc_kernel(variant_types_b, allele_frequencies_b,
                          jnp.reshape(haplotypes_bs, (B * S2,)),
                          jnp.reshape(priors_vc, (25,)),
                          jnp.reshape(snv_log_priors_rrra, (625,)))
    return jnp.reshape(out_flat, (B, 5))


# speedup vs baseline: 13.6611x; 1.1473x over previous
import functools

import jax
import jax.numpy as jnp
from jax import lax
from jax.experimental import pallas as pl
from jax.experimental.pallas import tpu as pltpu
from jax.experimental.pallas import tpu_sc as plsc

B = 524288
S2 = 26
NC = 2
NS = 16
NW = NC * NS
BPW = B // NW
CHUNK = 1024
NCHUNK = BPW // CHUNK

_LN2 = 0.69314718055994530942


def _vlog(x):
    bits = plsc.bitcast(x, jnp.int32)
    e = lax.shift_right_logical(bits, 23) - 127
    mbits = (bits & 0x007FFFFF) | 0x3F800000
    m = plsc.bitcast(mbits, jnp.float32)
    big = m > 1.41421356
    m = jnp.where(big, m * 0.5, m)
    ef = e.astype(jnp.float32) + jnp.where(big, 1.0, 0.0)
    t = m - 1.0
    z = t * t
    p = jnp.full((16,), 7.0376836292e-2, jnp.float32)
    p = p * t + (-1.1514610310e-1)
    p = p * t + 1.1676998740e-1
    p = p * t + (-1.2420140846e-1)
    p = p * t + 1.4249322787e-1
    p = p * t + (-1.6668057665e-1)
    p = p * t + 2.0000714765e-1
    p = p * t + (-2.4999993993e-1)
    p = p * t + 3.3333331174e-1
    y = t * (z * p) - 0.5 * z + t
    return y + ef * _LN2


def _sc_body(vt_hbm, af_hbm, hap_hbm, pri_hbm, snv_hbm, out_hbm,
             vt0, vt1, af0, af1, hap0, hap1, out0, out1, pri_v, snv_v,
             sin0, sin1, sout0, sout1):
    wid = lax.axis_index("s") * NC + lax.axis_index("c")
    base_w = wid * BPW
    pltpu.sync_copy(pri_hbm, pri_v)
    pltpu.sync_copy(snv_hbm, snv_v)
    lanes0 = lax.iota(jnp.int32, 16)

    bufs = ((vt0, af0, hap0, out0, sin0, sout0),
            (vt1, af1, hap1, out1, sin1, sout1))

    def issue_in(ci, b):
        vt_v, af_v, hap_v, _, sem, _ = bufs[b]
        cb = base_w + ci * CHUNK
        pltpu.async_copy(vt_hbm.at[pl.ds(cb, CHUNK)], vt_v, sem)
        pltpu.async_copy(af_hbm.at[pl.ds(cb, CHUNK)], af_v, sem)
        pltpu.async_copy(hap_hbm.at[pl.ds(cb * S2, CHUNK * S2)], hap_v, sem)

    def wait_in(b):
        vt_v, af_v, hap_v, _, sem, _ = bufs[b]
        pltpu.make_async_copy(vt_hbm.at[pl.ds(0, CHUNK)], vt_v, sem).wait()
        pltpu.make_async_copy(af_hbm.at[pl.ds(0, CHUNK)], af_v, sem).wait()
        pltpu.make_async_copy(hap_hbm.at[pl.ds(0, CHUNK * S2)], hap_v,
                              sem).wait()

    def issue_out(ci, b):
        out_v, sem = bufs[b][3], bufs[b][5]
        cb = base_w + ci * CHUNK
        pltpu.async_copy(out_v, out_hbm.at[pl.ds(cb * 5, CHUNK * 5)], sem)

    def wait_out(b):
        out_v, sem = bufs[b][3], bufs[b][5]
        pltpu.make_async_copy(out_v, out_hbm.at[pl.ds(0, CHUNK * 5)],
                              sem).wait()

    def compute_chunk(b):
        vt_v, af_v, hap_v, out_v = bufs[b][:4]

        @plsc.parallel_loop(0, CHUNK, 16, unroll=4)
        def _group(gb):
            lanes = lanes0 + gb
            vt = vt_v[pl.ds(gb, 16)]
            af = af_v[pl.ds(gb, 16)]
            hrow = lanes * S2
            i0 = plsc.load_gather(hap_v, [hrow + 5])
            i1 = plsc.load_gather(hap_v, [hrow + 6])
            i2 = plsc.load_gather(hap_v, [hrow + 7])
            i3 = plsc.load_gather(hap_v, [hrow + 19])
            snv = plsc.load_gather(snv_v, [((i0 * 5 + i1) * 5 + i2) * 5 + i3])
            vt5 = vt * 5
            p_som = plsc.load_gather(pri_v, [vt5])
            p_art = plsc.load_gather(pri_v, [vt5 + 1])
            p_nart = plsc.load_gather(pri_v, [vt5 + 4])
            is_snv = vt == 0
            c0 = jnp.where(is_snv, snv, p_som)
            u = 1.0 - af
            c3 = _vlog(1.0 - u * u)
            m = jnp.maximum(jnp.maximum(c0, p_art),
                            jnp.maximum(jnp.maximum(c3, p_nart), 0.0))
            s = (jnp.exp(c0 - m) + jnp.exp(p_art - m) + jnp.exp(-m)
                 + jnp.exp(c3 - m) + jnp.exp(p_nart - m))
            lse = m + _vlog(s)
            orow = lanes * 5
            plsc.store_scatter(out_v, [orow], c0 - lse)
            plsc.store_scatter(out_v, [orow + 1], p_art - lse)
            plsc.store_scatter(out_v, [orow + 2], -lse)
            plsc.store_scatter(out_v, [orow + 3], c3 - lse)
            plsc.store_scatter(out_v, [orow + 4], p_nart - lse)

    issue_in(0, 0)
    issue_in(1, 1)

    def pair_body(pi, carry):
        ci = pi * 2
        for half in range(2):
            b = half
            wait_in(b)

            @pl.when(pi > 0)
            def _():
                wait_out(b)

            compute_chunk(b)
            issue_out(ci + half, b)
            issue_in(ci + half + 2, b)
        return carry

    lax.fori_loop(0, NCHUNK // 2 - 1, pair_body, 0)
    for half in range(2):
        ci = NCHUNK - 2 + half
        wait_in(half)
        wait_out(half)
        compute_chunk(half)
        issue_out(ci, half)
    wait_out(0)
    wait_out(1)


_sc_kernel = functools.partial(
    pl.kernel,
    mesh=plsc.VectorSubcoreMesh(core_axis_name="c", subcore_axis_name="s"),
    out_type=jax.ShapeDtypeStruct((B * 5,), jnp.float32),
    scratch_types=[
        pltpu.VMEM((CHUNK,), jnp.int32),
        pltpu.VMEM((CHUNK,), jnp.int32),
        pltpu.VMEM((CHUNK,), jnp.float32),
        pltpu.VMEM((CHUNK,), jnp.float32),
        pltpu.VMEM((CHUNK * S2,), jnp.int32),
        pltpu.VMEM((CHUNK * S2,), jnp.int32),
        pltpu.VMEM((CHUNK * 5,), jnp.float32),
        pltpu.VMEM((CHUNK * 5,), jnp.float32),
        pltpu.VMEM((25,), jnp.float32),
        pltpu.VMEM((625,), jnp.float32),
        pltpu.SemaphoreType.DMA,
        pltpu.SemaphoreType.DMA,
        pltpu.SemaphoreType.DMA,
        pltpu.SemaphoreType.DMA,
    ],
    compiler_params=pltpu.CompilerParams(needs_layout_passes=False),
)(_sc_body)


def kernel(variant_types_b, allele_frequencies_b, haplotypes_bs,
           priors_vc, snv_log_priors_rrra):
    out_flat = _sc_kernel(variant_types_b, allele_frequencies_b,
                          jnp.reshape(haplotypes_bs, (B * S2,)),
                          jnp.reshape(priors_vc, (25,)),
                          jnp.reshape(snv_log_priors_rrra, (625,)))
    return jnp.reshape(out_flat, (B, 5))


# speedup vs baseline: 77.5721x; 5.6783x over previous
import functools

import jax
import jax.numpy as jnp
from jax import lax
from jax.experimental import pallas as pl
from jax.experimental.pallas import tpu as pltpu
from jax.experimental.pallas import tpu_sc as plsc

B = 524288
NC = 2
NS = 16
NW = NC * NS
BPW = B // NW
CHUNK = 4096
NCHUNK = BPW // CHUNK

_LN2 = 0.69314718055994530942


def _vlog(x):
    bits = plsc.bitcast(x, jnp.int32)
    e = lax.shift_right_logical(bits, 23) - 127
    mbits = (bits & 0x007FFFFF) | 0x3F800000
    m = plsc.bitcast(mbits, jnp.float32)
    big = m > 1.41421356
    m = jnp.where(big, m * 0.5, m)
    ef = e.astype(jnp.float32) + jnp.where(big, 1.0, 0.0)
    t = m - 1.0
    z = t * t
    p = jnp.full((16,), 7.0376836292e-2, jnp.float32)
    p = p * t + (-1.1514610310e-1)
    p = p * t + 1.1676998740e-1
    p = p * t + (-1.2420140846e-1)
    p = p * t + 1.4249322787e-1
    p = p * t + (-1.6668057665e-1)
    p = p * t + 2.0000714765e-1
    p = p * t + (-2.4999993993e-1)
    p = p * t + 3.3333331174e-1
    y = t * (z * p) - 0.5 * z + t
    return y + ef * _LN2


def _sc_body(vt_hbm, af_hbm, h0_hbm, h1_hbm, h2_hbm, h3_hbm,
             pri_hbm, snv_hbm,
             o0_hbm, o1_hbm, o2_hbm, o3_hbm, o4_hbm,
             ins0, ins1, outs0, outs1, pri_v, snv_v,
             sin0, sin1, sout0, sout1):
    wid = lax.axis_index("s") * NC + lax.axis_index("c")
    base_w = wid * BPW
    pltpu.sync_copy(pri_hbm, pri_v)
    pltpu.sync_copy(snv_hbm, snv_v)
    lanes0 = lax.iota(jnp.int32, 16)

    in_hbm = (vt_hbm, af_hbm, h0_hbm, h1_hbm, h2_hbm, h3_hbm)
    out_hbm = (o0_hbm, o1_hbm, o2_hbm, o3_hbm, o4_hbm)
    in_bufs = (ins0, ins1)
    out_bufs = (outs0, outs1)
    in_sems = (sin0, sin1)
    out_sems = (sout0, sout1)

    def issue_in(ci, b):
        cb = base_w + ci * CHUNK
        for k, src in enumerate(in_hbm):
            pltpu.async_copy(src.at[pl.ds(cb, CHUNK)], in_bufs[b][k],
                             in_sems[b])

    def wait_in(b):
        for k, src in enumerate(in_hbm):
            pltpu.make_async_copy(src.at[pl.ds(0, CHUNK)], in_bufs[b][k],
                                  in_sems[b]).wait()

    def issue_out(ci, b):
        cb = base_w + ci * CHUNK
        for k, dst in enumerate(out_hbm):
            pltpu.async_copy(out_bufs[b][k], dst.at[pl.ds(cb, CHUNK)],
                             out_sems[b])

    def wait_out(b):
        for k, dst in enumerate(out_hbm):
            pltpu.make_async_copy(out_bufs[b][k], dst.at[pl.ds(0, CHUNK)],
                                  out_sems[b]).wait()

    def compute_chunk(b):
        vt_v, af_v, h0_v, h1_v, h2_v, h3_v = in_bufs[b]
        o0_v, o1_v, o2_v, o3_v, o4_v = out_bufs[b]

        @plsc.parallel_loop(0, CHUNK, 16, unroll=4)
        def _group(gb):
            sl = pl.ds(gb, 16)
            vt = vt_v[sl]
            af = af_v[sl]
            i0 = h0_v[sl]
            i1 = h1_v[sl]
            i2 = h2_v[sl]
            i3 = h3_v[sl]
            snv = plsc.load_gather(snv_v, [((i0 * 5 + i1) * 5 + i2) * 5 + i3])
            vt5 = vt * 5
            p_som = plsc.load_gather(pri_v, [vt5])
            p_art = plsc.load_gather(pri_v, [vt5 + 1])
            p_nart = plsc.load_gather(pri_v, [vt5 + 4])
            is_snv = vt == 0
            c0 = jnp.where(is_snv, snv, p_som)
            u = 1.0 - af
            c3 = _vlog(1.0 - u * u)
            m = jnp.maximum(jnp.maximum(c0, p_art),
                            jnp.maximum(jnp.maximum(c3, p_nart), 0.0))
            s = (jnp.exp(c0 - m) + jnp.exp(p_art - m) + jnp.exp(-m)
                 + jnp.exp(c3 - m) + jnp.exp(p_nart - m))
            lse = m + _vlog(s)
            o0_v[sl] = c0 - lse
            o1_v[sl] = p_art - lse
            o2_v[sl] = -lse
            o3_v[sl] = c3 - lse
            o4_v[sl] = p_nart - lse

    issue_in(0, 0)
    issue_in(1, 1)
    for ci in range(NCHUNK):
        b = ci % 2
        wait_in(b)
        if ci >= 2:
            wait_out(b)
        compute_chunk(b)
        issue_out(ci, b)
        if ci + 2 < NCHUNK:
            issue_in(ci + 2, b)
    wait_out(0)
    wait_out(1)


_out1d = jax.ShapeDtypeStruct((B,), jnp.float32)
_sc_kernel = functools.partial(
    pl.kernel,
    mesh=plsc.VectorSubcoreMesh(core_axis_name="c", subcore_axis_name="s"),
    out_type=(_out1d, _out1d, _out1d, _out1d, _out1d),
    scratch_types=[
        tuple([pltpu.VMEM((CHUNK,), jnp.int32), pltpu.VMEM((CHUNK,), jnp.float32)]
              + [pltpu.VMEM((CHUNK,), jnp.int32)] * 4),
        tuple([pltpu.VMEM((CHUNK,), jnp.int32), pltpu.VMEM((CHUNK,), jnp.float32)]
              + [pltpu.VMEM((CHUNK,), jnp.int32)] * 4),
        tuple([pltpu.VMEM((CHUNK,), jnp.float32)] * 5),
        tuple([pltpu.VMEM((CHUNK,), jnp.float32)] * 5),
        pltpu.VMEM((25,), jnp.float32),
        pltpu.VMEM((625,), jnp.float32),
        pltpu.SemaphoreType.DMA,
        pltpu.SemaphoreType.DMA,
        pltpu.SemaphoreType.DMA,
        pltpu.SemaphoreType.DMA,
    ],
    compiler_params=pltpu.CompilerParams(needs_layout_passes=False),
)(_sc_body)


def kernel(variant_types_b, allele_frequencies_b, haplotypes_bs,
           priors_vc, snv_log_priors_rrra):
    h0 = haplotypes_bs[:, 5]
    h1 = haplotypes_bs[:, 6]
    h2 = haplotypes_bs[:, 7]
    h3 = haplotypes_bs[:, 19]
    outs = _sc_kernel(variant_types_b, allele_frequencies_b, h0, h1, h2, h3,
                      jnp.reshape(priors_vc, (25,)),
                      jnp.reshape(snv_log_priors_rrra, (625,)))
    return jnp.stack(outs, axis=1)


# speedup vs baseline: 77.7053x; 1.0017x over previous
import functools

import jax
import jax.numpy as jnp
from jax import lax
from jax.experimental import pallas as pl
from jax.experimental.pallas import tpu as pltpu
from jax.experimental.pallas import tpu_sc as plsc

B = 524288
NC = 2
NS = 16
NW = NC * NS
BPW = B // NW
CHUNK = 4096
NCHUNK = BPW // CHUNK

_LN2 = 0.69314718055994530942


def _vlog(x):
    bits = plsc.bitcast(x, jnp.int32)
    e = lax.shift_right_logical(bits, 23) - 127
    mbits = (bits & 0x007FFFFF) | 0x3F800000
    m = plsc.bitcast(mbits, jnp.float32)
    big = m > 1.41421356
    m = jnp.where(big, m * 0.5, m)
    ef = e.astype(jnp.float32) + jnp.where(big, 1.0, 0.0)
    t = m - 1.0
    z = t * t
    p = jnp.full((16,), 7.0376836292e-2, jnp.float32)
    p = p * t + (-1.1514610310e-1)
    p = p * t + 1.1676998740e-1
    p = p * t + (-1.2420140846e-1)
    p = p * t + 1.4249322787e-1
    p = p * t + (-1.6668057665e-1)
    p = p * t + 2.0000714765e-1
    p = p * t + (-2.4999993993e-1)
    p = p * t + 3.3333331174e-1
    y = t * (z * p) - 0.5 * z + t
    return y + ef * _LN2


def _sc_body(vt_hbm, af_hbm, h0_hbm, h1_hbm, h2_hbm, h3_hbm,
             pri_hbm, snv_hbm,
             o0_hbm, o1_hbm, o2_hbm, o3_hbm, o4_hbm,
             ins0, ins1, outs0, outs1, pri_v, snv_v,
             sin0, sin1, sout0, sout1):
    wid = lax.axis_index("s") * NC + lax.axis_index("c")
    base_w = wid * BPW

    in_hbm = (vt_hbm, af_hbm, h0_hbm, h1_hbm, h2_hbm, h3_hbm)
    out_hbm = (o0_hbm, o1_hbm, o2_hbm, o3_hbm, o4_hbm)
    in_bufs = (ins0, ins1)
    out_bufs = (outs0, outs1)
    in_sems = (sin0, sin1)
    out_sems = (sout0, sout1)

    def issue_in(ci, b):
        cb = base_w + ci * CHUNK
        for k, src in enumerate(in_hbm):
            pltpu.async_copy(src.at[pl.ds(cb, CHUNK)], in_bufs[b][k],
                             in_sems[b])

    def wait_in(b):
        for k, src in enumerate(in_hbm):
            pltpu.make_async_copy(src.at[pl.ds(0, CHUNK)], in_bufs[b][k],
                                  in_sems[b]).wait()

    def issue_out(ci, b):
        cb = base_w + ci * CHUNK
        for k, dst in enumerate(out_hbm):
            pltpu.async_copy(out_bufs[b][k], dst.at[pl.ds(cb, CHUNK)],
                             out_sems[b])

    def wait_out(b):
        for k, dst in enumerate(out_hbm):
            pltpu.make_async_copy(out_bufs[b][k], dst.at[pl.ds(0, CHUNK)],
                                  out_sems[b]).wait()

    def compute_chunk(b):
        vt_v, af_v, h0_v, h1_v, h2_v, h3_v = in_bufs[b]
        o0_v, o1_v, o2_v, o3_v, o4_v = out_bufs[b]

        @plsc.parallel_loop(0, CHUNK, 16, unroll=8)
        def _group(gb):
            sl = pl.ds(gb, 16)
            vt = vt_v[sl]
            af = af_v[sl]
            i0 = h0_v[sl]
            i1 = h1_v[sl]
            i2 = h2_v[sl]
            i3 = h3_v[sl]
            snv = plsc.load_gather(snv_v, [((i0 * 5 + i1) * 5 + i2) * 5 + i3])
            vt5 = vt * 5
            p_som = plsc.load_gather(pri_v, [vt5])
            p_art = plsc.load_gather(pri_v, [vt5 + 1])
            p_nart = plsc.load_gather(pri_v, [vt5 + 4])
            is_snv = vt == 0
            c0 = jnp.where(is_snv, snv, p_som)
            u = 1.0 - af
            g = 1.0 - u * u
            c3 = _vlog(g)
            m = jnp.maximum(jnp.maximum(c0, p_art),
                            jnp.maximum(p_nart, 0.0))
            em = jnp.exp(-m)
            s = (jnp.exp(c0 - m) + jnp.exp(p_art - m) + em
                 + g * em + jnp.exp(p_nart - m))
            lse = m + _vlog(s)
            o0_v[sl] = c0 - lse
            o1_v[sl] = p_art - lse
            o2_v[sl] = -lse
            o3_v[sl] = c3 - lse
            o4_v[sl] = p_nart - lse

    issue_in(0, 0)
    issue_in(1, 1)
    pltpu.sync_copy(pri_hbm, pri_v)
    pltpu.sync_copy(snv_hbm, snv_v)
    for ci in range(NCHUNK):
        b = ci % 2
        wait_in(b)
        if ci >= 2:
            wait_out(b)
        compute_chunk(b)
        issue_out(ci, b)
        if ci + 2 < NCHUNK:
            issue_in(ci + 2, b)
    wait_out(0)
    wait_out(1)


_out1d = jax.ShapeDtypeStruct((B,), jnp.float32)
_sc_kernel = functools.partial(
    pl.kernel,
    mesh=plsc.VectorSubcoreMesh(core_axis_name="c", subcore_axis_name="s"),
    out_type=(_out1d, _out1d, _out1d, _out1d, _out1d),
    scratch_types=[
        tuple([pltpu.VMEM((CHUNK,), jnp.int32), pltpu.VMEM((CHUNK,), jnp.float32)]
              + [pltpu.VMEM((CHUNK,), jnp.int32)] * 4),
        tuple([pltpu.VMEM((CHUNK,), jnp.int32), pltpu.VMEM((CHUNK,), jnp.float32)]
              + [pltpu.VMEM((CHUNK,), jnp.int32)] * 4),
        tuple([pltpu.VMEM((CHUNK,), jnp.float32)] * 5),
        tuple([pltpu.VMEM((CHUNK,), jnp.float32)] * 5),
        pltpu.VMEM((25,), jnp.float32),
        pltpu.VMEM((625,), jnp.float32),
        pltpu.SemaphoreType.DMA,
        pltpu.SemaphoreType.DMA,
        pltpu.SemaphoreType.DMA,
        pltpu.SemaphoreType.DMA,
    ],
    compiler_params=pltpu.CompilerParams(needs_layout_passes=False),
)(_sc_body)


def kernel(variant_types_b, allele_frequencies_b, haplotypes_bs,
           priors_vc, snv_log_priors_rrra):
    h0 = haplotypes_bs[:, 5]
    h1 = haplotypes_bs[:, 6]
    h2 = haplotypes_bs[:, 7]
    h3 = haplotypes_bs[:, 19]
    outs = _sc_kernel(variant_types_b, allele_frequencies_b, h0, h1, h2, h3,
                      jnp.reshape(priors_vc, (25,)),
                      jnp.reshape(snv_log_priors_rrra, (625,)))
    return jnp.stack(outs, axis=1)


# speedup vs baseline: 108.7854x; 1.4000x over previous
import functools

import jax
import jax.numpy as jnp
from jax import lax
from jax.experimental import pallas as pl
from jax.experimental.pallas import tpu as pltpu
from jax.experimental.pallas import tpu_sc as plsc

B = 524288
NC = 2
NS = 16
NW = NC * NS
BPW = B // NW
CHUNK = 4096
NCHUNK = BPW // CHUNK
CROWS = CHUNK // 128
BT = B // 128

_LN2 = 0.69314718055994530942


def _vlog(x):
    bits = plsc.bitcast(x, jnp.int32)
    e = lax.shift_right_logical(bits, 23) - 127
    mbits = (bits & 0x007FFFFF) | 0x3F800000
    m = plsc.bitcast(mbits, jnp.float32)
    big = m > 1.41421356
    m = jnp.where(big, m * 0.5, m)
    ef = e.astype(jnp.float32) + jnp.where(big, 1.0, 0.0)
    t = m - 1.0
    z = t * t
    p = jnp.full((16,), 7.0376836292e-2, jnp.float32)
    p = p * t + (-1.1514610310e-1)
    p = p * t + 1.1676998740e-1
    p = p * t + (-1.2420140846e-1)
    p = p * t + 1.4249322787e-1
    p = p * t + (-1.6668057665e-1)
    p = p * t + 2.0000714765e-1
    p = p * t + (-2.4999993993e-1)
    p = p * t + 3.3333331174e-1
    y = t * (z * p) - 0.5 * z + t
    return y + ef * _LN2


def _sc_body(vt_hbm, af_hbm, gi_hbm, pri_hbm, snv_hbm, out_hbm,
             ins0, ins1, outs0, outs1, pri_v, snv_v,
             sin0, sin1, sout0, sout1):
    wid = lax.axis_index("s") * NC + lax.axis_index("c")
    base_w = wid * BPW

    in_hbm = (vt_hbm, af_hbm, gi_hbm)
    in_bufs = (ins0, ins1)
    out_bufs = (outs0, outs1)
    in_sems = (sin0, sin1)
    out_sems = (sout0, sout1)

    def issue_in(ci, b):
        cb = base_w + ci * CHUNK
        for k, src in enumerate(in_hbm):
            pltpu.async_copy(src.at[pl.ds(cb, CHUNK)], in_bufs[b][k],
                             in_sems[b])

    def wait_in(b):
        for k, src in enumerate(in_hbm):
            pltpu.make_async_copy(src.at[pl.ds(0, CHUNK)], in_bufs[b][k],
                                  in_sems[b]).wait()

    def issue_out(ci, b):
        ct0 = (base_w + ci * CHUNK) // 128
        for c in range(5):
            pltpu.async_copy(out_bufs[b][c],
                             out_hbm.at[pl.ds(ct0, CROWS), c], out_sems[b])

    def wait_out(b):
        for c in range(5):
            pltpu.make_async_copy(out_bufs[b][c],
                                  out_hbm.at[pl.ds(0, CROWS), c],
                                  out_sems[b]).wait()

    def compute_chunk(b):
        vt_v, af_v, gi_v = in_bufs[b]
        o0_v, o1_v, o2_v, o3_v, o4_v = out_bufs[b]

        @plsc.parallel_loop(0, CHUNK, 16, unroll=8)
        def _group(gb):
            sl = pl.ds(gb, 16)
            vt = vt_v[sl]
            af = af_v[sl]
            gi = gi_v[sl]
            snv = plsc.load_gather(snv_v, [gi])
            vt5 = vt * 5
            p_som = plsc.load_gather(pri_v, [vt5])
            p_art = plsc.load_gather(pri_v, [vt5 + 1])
            p_nart = plsc.load_gather(pri_v, [vt5 + 4])
            is_snv = vt == 0
            c0 = jnp.where(is_snv, snv, p_som)
            u = 1.0 - af
            g = 1.0 - u * u
            c3 = _vlog(g)
            m = jnp.maximum(jnp.maximum(c0, p_art),
                            jnp.maximum(p_nart, 0.0))
            em = jnp.exp(-m)
            s = (jnp.exp(c0 - m) + jnp.exp(p_art - m) + em
                 + g * em + jnp.exp(p_nart - m))
            lse = m + _vlog(s)
            row = lax.shift_right_logical(gb, 7)
            csl = pl.ds(gb & 127, 16)
            o0_v[row, csl] = c0 - lse
            o1_v[row, csl] = p_art - lse
            o2_v[row, csl] = -lse
            o3_v[row, csl] = c3 - lse
            o4_v[row, csl] = p_nart - lse

    issue_in(0, 0)
    issue_in(1, 1)
    pltpu.sync_copy(pri_hbm, pri_v)
    pltpu.sync_copy(snv_hbm, snv_v)
    for ci in range(NCHUNK):
        b = ci % 2
        wait_in(b)
        if ci >= 2:
            wait_out(b)
        compute_chunk(b)
        issue_out(ci, b)
        if ci + 2 < NCHUNK:
            issue_in(ci + 2, b)
    wait_out(0)
    wait_out(1)


_sc_kernel = functools.partial(
    pl.kernel,
    mesh=plsc.VectorSubcoreMesh(core_axis_name="c", subcore_axis_name="s"),
    out_type=jax.ShapeDtypeStruct((BT, 8, 128), jnp.float32),
    scratch_types=[
        (pltpu.VMEM((CHUNK,), jnp.int32), pltpu.VMEM((CHUNK,), jnp.float32),
         pltpu.VMEM((CHUNK,), jnp.int32)),
        (pltpu.VMEM((CHUNK,), jnp.int32), pltpu.VMEM((CHUNK,), jnp.float32),
         pltpu.VMEM((CHUNK,), jnp.int32)),
        tuple([pltpu.VMEM((CROWS, 128), jnp.float32)] * 5),
        tuple([pltpu.VMEM((CROWS, 128), jnp.float32)] * 5),
        pltpu.VMEM((25,), jnp.float32),
        pltpu.VMEM((625,), jnp.float32),
        pltpu.SemaphoreType.DMA,
        pltpu.SemaphoreType.DMA,
        pltpu.SemaphoreType.DMA,
        pltpu.SemaphoreType.DMA,
    ],
    compiler_params=pltpu.CompilerParams(needs_layout_passes=False),
)(_sc_body)


def kernel(variant_types_b, allele_frequencies_b, haplotypes_bs,
           priors_vc, snv_log_priors_rrra):
    h0 = haplotypes_bs[:, 5]
    h1 = haplotypes_bs[:, 6]
    h2 = haplotypes_bs[:, 7]
    h3 = haplotypes_bs[:, 19]
    gidx = ((h0 * 5 + h1) * 5 + h2) * 5 + h3
    out3 = _sc_kernel(variant_types_b, allele_frequencies_b, gidx,
                      jnp.reshape(priors_vc, (25,)),
                      jnp.reshape(snv_log_priors_rrra, (625,)))
    res = lax.slice(out3, (0, 0, 0), (BT, 5, 128))
    return jnp.reshape(jnp.transpose(res, (0, 2, 1)), (B, 5))


# speedup vs baseline: 144.3650x; 1.3271x over previous
import functools

import jax
import jax.numpy as jnp
from jax import lax
from jax.experimental import pallas as pl
from jax.experimental.pallas import tpu as pltpu
from jax.experimental.pallas import tpu_sc as plsc

B = 524288
NC = 2
NS = 16
NW = NC * NS
BPW = B // NW
CHUNK = 4096
NCHUNK = BPW // CHUNK
CROWS = CHUNK // 128
BT = B // 128
HCOLS = (5, 6, 7, 19)

_LN2 = 0.69314718055994530942


def _vlog(x):
    bits = plsc.bitcast(x, jnp.int32)
    e = lax.shift_right_logical(bits, 23) - 127
    mbits = (bits & 0x007FFFFF) | 0x3F800000
    m = plsc.bitcast(mbits, jnp.float32)
    big = m > 1.41421356
    m = jnp.where(big, m * 0.5, m)
    ef = e.astype(jnp.float32) + jnp.where(big, 1.0, 0.0)
    t = m - 1.0
    z = t * t
    p = jnp.full((16,), 7.0376836292e-2, jnp.float32)
    p = p * t + (-1.1514610310e-1)
    p = p * t + 1.1676998740e-1
    p = p * t + (-1.2420140846e-1)
    p = p * t + 1.4249322787e-1
    p = p * t + (-1.6668057665e-1)
    p = p * t + 2.0000714765e-1
    p = p * t + (-2.4999993993e-1)
    p = p * t + 3.3333331174e-1
    y = t * (z * p) - 0.5 * z + t
    return y + ef * _LN2


def _sc_body(vt_hbm, af_hbm, hap_hbm, pri_hbm, snv_hbm, out_hbm,
             ins0, ins1, outs0, outs1, pri_v, snv_v,
             sin0, sin1, sout0, sout1):
    wid = lax.axis_index("s") * NC + lax.axis_index("c")
    base_w = wid * BPW

    in_bufs = (ins0, ins1)
    out_bufs = (outs0, outs1)
    in_sems = (sin0, sin1)
    out_sems = (sout0, sout1)

    def issue_in(ci, b):
        cb = base_w + ci * CHUNK
        sl = pl.ds(cb, CHUNK)
        pltpu.async_copy(vt_hbm.at[sl], in_bufs[b][0], in_sems[b])
        pltpu.async_copy(af_hbm.at[sl], in_bufs[b][1], in_sems[b])
        for k, c in enumerate(HCOLS):
            pltpu.async_copy(hap_hbm.at[c, sl], in_bufs[b][2 + k],
                             in_sems[b])

    def wait_in(b):
        sl = pl.ds(0, CHUNK)
        pltpu.make_async_copy(vt_hbm.at[sl], in_bufs[b][0], in_sems[b]).wait()
        pltpu.make_async_copy(af_hbm.at[sl], in_bufs[b][1], in_sems[b]).wait()
        for k, c in enumerate(HCOLS):
            pltpu.make_async_copy(hap_hbm.at[c, sl], in_bufs[b][2 + k],
                                  in_sems[b]).wait()

    def issue_out(ci, b):
        ct0 = (base_w + ci * CHUNK) // 128
        for c in range(5):
            pltpu.async_copy(out_bufs[b][c],
                             out_hbm.at[pl.ds(ct0, CROWS), c], out_sems[b])

    def wait_out(b):
        for c in range(5):
            pltpu.make_async_copy(out_bufs[b][c],
                                  out_hbm.at[pl.ds(0, CROWS), c],
                                  out_sems[b]).wait()

    def compute_chunk(b):
        vt_v, af_v, h0_v, h1_v, h2_v, h3_v = in_bufs[b]
        o0_v, o1_v, o2_v, o3_v, o4_v = out_bufs[b]

        @plsc.parallel_loop(0, CHUNK, 16, unroll=8)
        def _group(gb):
            sl = pl.ds(gb, 16)
            vt = vt_v[sl]
            af = af_v[sl]
            gi = ((h0_v[sl] * 5 + h1_v[sl]) * 5 + h2_v[sl]) * 5 + h3_v[sl]
            snv = plsc.load_gather(snv_v, [gi])
            vt5 = vt * 5
            p_som = plsc.load_gather(pri_v, [vt5])
            p_art = plsc.load_gather(pri_v, [vt5 + 1])
            p_nart = plsc.load_gather(pri_v, [vt5 + 4])
            is_snv = vt == 0
            c0 = jnp.where(is_snv, snv, p_som)
            u = 1.0 - af
            g = 1.0 - u * u
            c3 = _vlog(g)
            m = jnp.maximum(jnp.maximum(c0, p_art),
                            jnp.maximum(p_nart, 0.0))
            em = jnp.exp(-m)
            s = (jnp.exp(c0 - m) + jnp.exp(p_art - m) + em
                 + g * em + jnp.exp(p_nart - m))
            lse = m + _vlog(s)
            row = lax.shift_right_logical(gb, 7)
            csl = pl.ds(gb & 127, 16)
            o0_v[row, csl] = c0 - lse
            o1_v[row, csl] = p_art - lse
            o2_v[row, csl] = -lse
            o3_v[row, csl] = c3 - lse
            o4_v[row, csl] = p_nart - lse

    issue_in(0, 0)
    issue_in(1, 1)
    pltpu.sync_copy(pri_hbm, pri_v)
    pltpu.sync_copy(snv_hbm, snv_v)
    for ci in range(NCHUNK):
        b = ci % 2
        wait_in(b)
        if ci >= 2:
            wait_out(b)
        compute_chunk(b)
        issue_out(ci, b)
        if ci + 2 < NCHUNK:
            issue_in(ci + 2, b)
    wait_out(0)
    wait_out(1)


_sc_kernel = functools.partial(
    pl.kernel,
    mesh=plsc.VectorSubcoreMesh(core_axis_name="c", subcore_axis_name="s"),
    out_type=jax.ShapeDtypeStruct((BT, 8, 128), jnp.float32),
    scratch_types=[
        tuple([pltpu.VMEM((CHUNK,), jnp.int32), pltpu.VMEM((CHUNK,), jnp.float32)]
              + [pltpu.VMEM((CHUNK,), jnp.int32)] * 4),
        tuple([pltpu.VMEM((CHUNK,), jnp.int32), pltpu.VMEM((CHUNK,), jnp.float32)]
              + [pltpu.VMEM((CHUNK,), jnp.int32)] * 4),
        tuple([pltpu.VMEM((CROWS, 128), jnp.float32)] * 5),
        tuple([pltpu.VMEM((CROWS, 128), jnp.float32)] * 5),
        pltpu.VMEM((25,), jnp.float32),
        pltpu.VMEM((625,), jnp.float32),
        pltpu.SemaphoreType.DMA,
        pltpu.SemaphoreType.DMA,
        pltpu.SemaphoreType.DMA,
        pltpu.SemaphoreType.DMA,
    ],
    compiler_params=pltpu.CompilerParams(needs_layout_passes=False),
)(_sc_body)


def kernel(variant_types_b, allele_frequencies_b, haplotypes_bs,
           priors_vc, snv_log_priors_rrra):
    hap_t = jnp.transpose(haplotypes_bs)
    out3 = _sc_kernel(variant_types_b, allele_frequencies_b, hap_t,
                      jnp.reshape(priors_vc, (25,)),
                      jnp.reshape(snv_log_priors_rrra, (625,)))
    res = lax.slice(out3, (0, 0, 0), (BT, 5, 128))
    return jnp.reshape(jnp.transpose(res, (0, 2, 1)), (B, 5))
